# Initial kernel scaffold; baseline (speedup 1.0000x reference)
import jax
import jax.numpy as jnp
from jax.experimental import pallas as pl


def kernel(x, edge_index, batch, W1l, W1r, b1, W2l, W2r, b2, W3l, W3r, b3, gamma, beta):
    raise NotImplementedError("write your pallas kernel here")


# speedup vs baseline: 3.8955x; 3.8955x over previous
import functools

import jax
import jax.numpy as jnp
from jax import lax
from jax.experimental import pallas as pl
from jax.experimental.pallas import tpu as pltpu
from jax.experimental.pallas import tpu_sc as plsc

N = 50000
E = 800000
DIN = 128
DH = 256
DOUT = 128
G = 16

NC, NS, L = 2, 16, 16
NPAD = 50176
RPS = NPAD // NS
EPAD = 819200
C = 32
WA = 256
NWA = (EPAD // NS) // WA
WD = 512
NWD = (EPAD // (NC * NS)) // WD
WS = 512
NWS = (EPAD // (NC * NS)) // WS
RB = 512
NBLK = NPAD // RB

_MESH = plsc.VectorSubcoreMesh(core_axis_name="c", subcore_axis_name="s")
_CP = pltpu.CompilerParams(needs_layout_passes=False, use_tc_tiling_on_sc=False)


def _fill_zeros_2d(ref, rows, width):
    @pl.loop(0, rows)
    def _(i):
        @pl.loop(0, width // L)
        def _(j):
            ref[i, pl.ds(j * L, L)] = jnp.zeros((L,), jnp.float32)


def _fill_1d(ref, n, val):
    @pl.loop(0, n // L)
    def _(i):
        ref[pl.ds(i * L, L)] = jnp.full((L,), val, jnp.float32)


def _sc_deg(dstp):
    @functools.partial(
        pl.kernel, mesh=_MESH, compiler_params=_CP,
        out_type=jax.ShapeDtypeStruct((NC, NPAD), jnp.float32),
        scratch_types=[
            pltpu.VMEM((WD,), jnp.int32),
            pltpu.VMEM((WD,), jnp.int32),
            pltpu.VMEM((WD,), jnp.float32),
            pltpu.VMEM((RPS,), jnp.float32),
            pltpu.VMEM_SHARED((NPAD,), jnp.float32),
        ],
    )
    def k(dst_h, deg_h, idd0, idd1, ones, zb1, dacc):
        cid = lax.axis_index("c")
        sid = lax.axis_index("s")
        wid = sid * NC + cid
        r0 = sid * RPS
        base = wid * (EPAD // (NC * NS))

        _fill_1d(ones, WD, 1.0)
        _fill_1d(zb1, RPS, 0.0)
        pltpu.sync_copy(zb1, dacc.at[pl.ds(r0, RPS)])
        plsc.subcore_barrier()

        @pl.loop(0, NWD, step=2)
        def _(w):
            pltpu.sync_copy(dst_h.at[pl.ds(base + w * WD, WD)], idd0)
            pltpu.sync_copy(ones, dacc.at[idd0], add=True)
            pltpu.sync_copy(dst_h.at[pl.ds(base + (w + 1) * WD, WD)], idd1)
            pltpu.sync_copy(ones, dacc.at[idd1], add=True)
        plsc.subcore_barrier()

        pltpu.sync_copy(dacc.at[pl.ds(r0, RPS)], deg_h.at[cid, pl.ds(r0, RPS)])

    return k(dstp)


def _sc_agg(xchunks, srcp, dstp):
    nchunk = len(xchunks)
    out_t = tuple(jax.ShapeDtypeStruct((NPAD, C), jnp.float32)
                  for _ in range(nchunk))

    @functools.partial(
        pl.kernel, mesh=_MESH, compiler_params=_CP,
        out_type=out_t,
        scratch_types=[
            pltpu.VMEM((WA,), jnp.int32),
            pltpu.VMEM((WA,), jnp.int32),
            pltpu.VMEM((WA,), jnp.int32),
            pltpu.VMEM((WA,), jnp.int32),
            pltpu.VMEM((WA, C), jnp.float32),
            pltpu.VMEM((WA, C), jnp.float32),
            pltpu.VMEM((112, C), jnp.float32),
            pltpu.VMEM_SHARED((NPAD, C), jnp.float32),
            pltpu.SemaphoreType.DMA,
            pltpu.SemaphoreType.DMA,
        ],
    )
    def k(*refs):
        x_hs = refs[:nchunk]
        src_h, dst_h = refs[nchunk], refs[nchunk + 1]
        out_hs = refs[nchunk + 2:2 * nchunk + 2]
        (ids0, ids1, idd0, idd1, rows0, rows1, zb, acc,
         sem0, sem1) = refs[2 * nchunk + 2:]
        cid = lax.axis_index("c")
        sid = lax.axis_index("s")
        r0 = sid * RPS
        base = sid * (EPAD // NS)

        _fill_zeros_2d(zb, 112, C)

        def windows(x_h):
            @pl.loop(0, NWA, step=2)
            def _(w):
                pltpu.sync_copy(src_h.at[pl.ds(base + w * WA, WA)], ids0)
                cp0 = pltpu.async_copy(x_h.at[ids0], rows0, sem0)
                pltpu.sync_copy(src_h.at[pl.ds(base + (w + 1) * WA, WA)], ids1)
                cp1 = pltpu.async_copy(x_h.at[ids1], rows1, sem1)
                pltpu.sync_copy(dst_h.at[pl.ds(base + w * WA, WA)], idd0)
                pltpu.sync_copy(dst_h.at[pl.ds(base + (w + 1) * WA, WA)], idd1)
                cp0.wait()
                pltpu.sync_copy(rows0, acc.at[idd0], add=True)
                cp1.wait()
                pltpu.sync_copy(rows1, acc.at[idd1], add=True)

        for kc in range(nchunk // 2):
            @pl.loop(0, RPS // 112)
            def _(kk):
                pltpu.sync_copy(zb, acc.at[pl.ds(r0 + kk * 112, 112)])
            plsc.subcore_barrier()

            @pl.when(cid == 0)
            def _():
                windows(x_hs[2 * kc])

            @pl.when(cid == 1)
            def _():
                windows(x_hs[2 * kc + 1])
            plsc.subcore_barrier()

            @pl.when(cid == 0)
            def _():
                pltpu.sync_copy(acc.at[pl.ds(r0, RPS)],
                                out_hs[2 * kc].at[pl.ds(r0, RPS)])

            @pl.when(cid == 1)
            def _():
                pltpu.sync_copy(acc.at[pl.ds(r0, RPS)],
                                out_hs[2 * kc + 1].at[pl.ds(r0, RPS)])

    return k(*xchunks, srcp, dstp)


def _sc_sagg(wrow, srcp, dstp):
    @functools.partial(
        pl.kernel, mesh=_MESH, compiler_params=_CP,
        out_type=jax.ShapeDtypeStruct((NC, NPAD, G), jnp.float32),
        scratch_types=[
            pltpu.VMEM((WS,), jnp.int32),
            pltpu.VMEM((WS,), jnp.int32),
            pltpu.VMEM((WS,), jnp.int32),
            pltpu.VMEM((WS,), jnp.int32),
            pltpu.VMEM((WS, G), jnp.float32),
            pltpu.VMEM((WS, G), jnp.float32),
            pltpu.VMEM((112, G), jnp.float32),
            pltpu.VMEM_SHARED((NPAD, G), jnp.float32),
            pltpu.SemaphoreType.DMA,
            pltpu.SemaphoreType.DMA,
        ],
    )
    def k(w_h, src_h, dst_h, s_h, ids0, ids1, idd0, idd1, rows0, rows1,
          zb, acc, sem0, sem1):
        cid = lax.axis_index("c")
        sid = lax.axis_index("s")
        wid = sid * NC + cid
        r0 = sid * RPS
        base = wid * (EPAD // (NC * NS))

        _fill_zeros_2d(zb, 112, G)

        @pl.loop(0, RPS // 112)
        def _(kk):
            pltpu.sync_copy(zb, acc.at[pl.ds(r0 + kk * 112, 112)])
        plsc.subcore_barrier()

        @pl.loop(0, NWS, step=2)
        def _(w):
            pltpu.sync_copy(dst_h.at[pl.ds(base + w * WS, WS)], idd0)
            cp0 = pltpu.async_copy(w_h.at[idd0], rows0, sem0)
            pltpu.sync_copy(dst_h.at[pl.ds(base + (w + 1) * WS, WS)], idd1)
            cp1 = pltpu.async_copy(w_h.at[idd1], rows1, sem1)
            pltpu.sync_copy(src_h.at[pl.ds(base + w * WS, WS)], ids0)
            pltpu.sync_copy(src_h.at[pl.ds(base + (w + 1) * WS, WS)], ids1)
            cp0.wait()
            pltpu.sync_copy(rows0, acc.at[ids0], add=True)
            cp1.wait()
            pltpu.sync_copy(rows1, acc.at[ids1], add=True)
        plsc.subcore_barrier()

        pltpu.sync_copy(acc.at[pl.ds(r0, RPS)],
                        s_h.at[cid, pl.ds(r0, RPS)])

    return k(wrow, srcp, dstp)


def _prep_body(deg_r, batch_r, winv_r, wrow_r):
    d = deg_r[0] + deg_r[1]
    winv = 1.0 / jnp.maximum(d, 1.0)
    winv_r[...] = winv
    giota = lax.broadcasted_iota(jnp.int32, (RB, G), 1)
    oh = jnp.where(batch_r[...] == giota, 1.0, 0.0)
    wrow_r[...] = oh * winv


def _tc_prep(deg2, batch_p):
    return pl.pallas_call(
        _prep_body,
        grid=(NBLK,),
        in_specs=[
            pl.BlockSpec((NC, RB, 1), lambda i: (0, i, 0)),
            pl.BlockSpec((RB, 1), lambda i: (i, 0)),
        ],
        out_specs=[
            pl.BlockSpec((RB, 1), lambda i: (i, 0)),
            pl.BlockSpec((RB, G), lambda i: (i, 0)),
        ],
        out_shape=(
            jax.ShapeDtypeStruct((NPAD, 1), jnp.float32),
            jax.ShapeDtypeStruct((NPAD, G), jnp.float32),
        ),
    )(deg2, batch_p)


def _lin1_body(*refs):
    a = refs[:DIN // C]
    x_r, winv_r, wl_r, wr_r, b_r = refs[DIN // C:DIN // C + 5]
    outs = refs[DIN // C + 5:]
    agg = jnp.concatenate([r[...] for r in a], axis=1)
    h = (jnp.dot(agg * winv_r[...], wl_r[...], preferred_element_type=jnp.float32)
         + jnp.dot(x_r[...], wr_r[...], preferred_element_type=jnp.float32)
         + b_r[...])
    h = jnp.maximum(h, 0.0)
    for c in range(DH // C):
        outs[c][...] = h[:, c * C:(c + 1) * C]


def _tc_lin1(aggs, x, winv2d, wlT, wrT, b2d):
    spec_c = pl.BlockSpec((RB, C), lambda i: (i, 0))
    full = lambda s: pl.BlockSpec(s, lambda i: tuple(0 for _ in s))
    return pl.pallas_call(
        _lin1_body,
        grid=(NBLK,),
        in_specs=[spec_c] * (DIN // C) + [
            pl.BlockSpec((RB, DIN), lambda i: (i, 0)),
            pl.BlockSpec((RB, 1), lambda i: (i, 0)),
            full((DIN, DH)), full((DIN, DH)), full((1, DH)),
        ],
        out_specs=[spec_c] * (DH // C),
        out_shape=tuple(jax.ShapeDtypeStruct((NPAD, C), jnp.float32)
                        for _ in range(DH // C)),
    )(*aggs, x, winv2d, wlT, wrT, b2d)


def _lin2_body(*refs):
    nc = DH // C
    aggs, h1s = refs[:nc], refs[nc:2 * nc]
    winv_r, wl_r, wr_r, b_r, out_r = refs[2 * nc:]
    agg = jnp.concatenate([r[...] for r in aggs], axis=1)
    h1 = jnp.concatenate([r[...] for r in h1s], axis=1)
    h = (jnp.dot(agg * winv_r[...], wl_r[...], preferred_element_type=jnp.float32)
         + jnp.dot(h1, wr_r[...], preferred_element_type=jnp.float32)
         + b_r[...])
    out_r[...] = jnp.maximum(h, 0.0)


def _tc_lin2(agg2c, h1c, winv2d, wlT, wrT, b2d):
    spec_c = pl.BlockSpec((RB, C), lambda i: (i, 0))
    full = lambda s: pl.BlockSpec(s, lambda i: tuple(0 for _ in s))
    return pl.pallas_call(
        _lin2_body,
        grid=(NBLK,),
        in_specs=[spec_c] * (2 * (DH // C)) + [
            pl.BlockSpec((RB, 1), lambda i: (i, 0)),
            full((DH, DH)), full((DH, DH)), full((1, DH)),
        ],
        out_specs=pl.BlockSpec((RB, DH), lambda i: (i, 0)),
        out_shape=jax.ShapeDtypeStruct((NPAD, DH), jnp.float32),
    )(*agg2c, *h1c, winv2d, wlT, wrT, b2d)


def _pool_body(h2_r, s_r, batch_r, wl_r, wr_r, b_r, g_r, be_r, out_r,
               ts_ref, tp_ref, cnt_ref):
    i = pl.program_id(0)

    @pl.when(i == 0)
    def _():
        ts_ref[...] = jnp.zeros((G, DH), jnp.float32)
        tp_ref[...] = jnp.zeros((G, DH), jnp.float32)
        cnt_ref[...] = jnp.zeros((1, G), jnp.float32)

    rows = i * RB + lax.broadcasted_iota(jnp.int32, (RB, G), 0)
    valid = rows < N
    s_blk = jnp.where(valid, s_r[0] + s_r[1], 0.0)
    giota = lax.broadcasted_iota(jnp.int32, (RB, G), 1)
    oh = jnp.where((batch_r[...] == giota) & valid, 1.0, 0.0)
    h2 = h2_r[...]
    dn = (((0,), (0,)), ((), ()))
    ts_ref[...] += lax.dot_general(s_blk, h2, dn,
                                   preferred_element_type=jnp.float32)
    tp_ref[...] += lax.dot_general(oh, h2, dn,
                                   preferred_element_type=jnp.float32)
    cnt_ref[...] += jnp.sum(oh, axis=0)[None, :]

    @pl.when(i == NBLK - 1)
    def _():
        cnt = cnt_ref[0, :]
        sums = (jnp.dot(ts_ref[...], wl_r[...],
                        preferred_element_type=jnp.float32)
                + jnp.dot(tp_ref[...], wr_r[...],
                          preferred_element_type=jnp.float32)
                + cnt[:, None] * b_r[...])
        pooled = sums / jnp.maximum(cnt, 1.0)[:, None]
        mu = jnp.mean(pooled, axis=1, keepdims=True)
        var = jnp.mean((pooled - mu) ** 2, axis=1, keepdims=True)
        normed = (pooled - mu) / jnp.sqrt(var + 1e-5)
        out_r[...] = normed * g_r[...] + be_r[...]


def _tc_pool(h2, s2, batch_p, w3lT, w3rT, b3_2d, gamma2d, beta2d):
    full = lambda s: pl.BlockSpec(s, lambda i: tuple(0 for _ in s))
    return pl.pallas_call(
        _pool_body,
        grid=(NBLK,),
        in_specs=[
            pl.BlockSpec((RB, DH), lambda i: (i, 0)),
            pl.BlockSpec((NC, RB, G), lambda i: (0, i, 0)),
            pl.BlockSpec((RB, 1), lambda i: (i, 0)),
            full((DH, DOUT)), full((DH, DOUT)), full((1, DOUT)),
            full((1, DOUT)), full((1, DOUT)),
        ],
        out_specs=full((G, DOUT)),
        out_shape=jax.ShapeDtypeStruct((G, DOUT), jnp.float32),
        scratch_shapes=[
            pltpu.VMEM((G, DH), jnp.float32),
            pltpu.VMEM((G, DH), jnp.float32),
            pltpu.VMEM((1, G), jnp.float32),
        ],
    )(h2, s2, batch_p, w3lT, w3rT, b3_2d, gamma2d, beta2d)


def kernel(x, edge_index, batch, W1l, W1r, b1, W2l, W2r, b2,
           W3l, W3r, b3, gamma, beta):
    src = edge_index[0]
    dst = edge_index[1]
    pad_e = EPAD - E
    srcp = jnp.concatenate([src, jnp.full((pad_e,), N, jnp.int32)])
    dstp = jnp.concatenate([dst, jnp.full((pad_e,), N, jnp.int32)])
    batch_p = jnp.pad(batch, (0, NPAD - N),
                      constant_values=G).reshape(NPAD, 1)
    xp = jnp.pad(x, ((0, NPAD - N), (0, 0)))
    xchunks = [xp[:, c * C:(c + 1) * C] for c in range(DIN // C)]

    deg2 = _sc_deg(dstp).reshape(NC, NPAD, 1)
    winv2d, wrow = _tc_prep(deg2, batch_p)

    agg1 = _sc_agg(xchunks, srcp, dstp)
    h1c = _tc_lin1(agg1, xp, winv2d, W1l.T, W1r.T, b1.reshape(1, DH))

    s2 = _sc_sagg(wrow, srcp, dstp)

    agg2 = _sc_agg(list(h1c), srcp, dstp)
    h2 = _tc_lin2(agg2, h1c, winv2d, W2l.T, W2r.T, b2.reshape(1, DH))

    return _tc_pool(h2, s2, batch_p, W3l.T, W3r.T, b3.reshape(1, DOUT),
                    gamma.reshape(1, DOUT), beta.reshape(1, DOUT))


# speedup vs baseline: 4.0546x; 1.0408x over previous
import functools

import jax
import jax.numpy as jnp
from jax import lax
from jax.experimental import pallas as pl
from jax.experimental.pallas import tpu as pltpu
from jax.experimental.pallas import tpu_sc as plsc

N = 50000
E = 800000
DIN = 128
DH = 256
DOUT = 128
G = 16

NC, NS, L = 2, 16, 16
NPAD = 50176
RPS = NPAD // NS
EPAD = 819200
C = 32
WA = 256
KGA = 10
GRPA = (EPAD // NS) // (KGA * WA)
WD = 512
NWD = (EPAD // (NC * NS)) // WD
WS = 512
KGS = 10
GRPS = (EPAD // (NC * NS)) // (KGS * WS)
RB = 512
NBLK = NPAD // RB

_MESH = plsc.VectorSubcoreMesh(core_axis_name="c", subcore_axis_name="s")
_CP = pltpu.CompilerParams(needs_layout_passes=False, use_tc_tiling_on_sc=False)


def _fill_zeros_2d(ref, rows, width):
    @pl.loop(0, rows)
    def _(i):
        @pl.loop(0, width // L)
        def _(j):
            ref[i, pl.ds(j * L, L)] = jnp.zeros((L,), jnp.float32)


def _fill_1d(ref, n, val):
    @pl.loop(0, n // L)
    def _(i):
        ref[pl.ds(i * L, L)] = jnp.full((L,), val, jnp.float32)


def _sc_deg(dstp):
    @functools.partial(
        pl.kernel, mesh=_MESH, compiler_params=_CP,
        out_type=jax.ShapeDtypeStruct((NC, NPAD), jnp.float32),
        scratch_types=[
            pltpu.VMEM((WD,), jnp.int32),
            pltpu.VMEM((WD,), jnp.int32),
            pltpu.VMEM((WD,), jnp.float32),
            pltpu.VMEM((RPS,), jnp.float32),
            pltpu.VMEM_SHARED((NPAD,), jnp.float32),
        ],
    )
    def k(dst_h, deg_h, idd0, idd1, ones, zb1, dacc):
        cid = lax.axis_index("c")
        sid = lax.axis_index("s")
        wid = sid * NC + cid
        r0 = sid * RPS
        base = wid * (EPAD // (NC * NS))

        _fill_1d(ones, WD, 1.0)
        _fill_1d(zb1, RPS, 0.0)
        pltpu.sync_copy(zb1, dacc.at[pl.ds(r0, RPS)])
        plsc.subcore_barrier()

        @pl.loop(0, NWD, step=2)
        def _(w):
            pltpu.sync_copy(dst_h.at[pl.ds(base + w * WD, WD)], idd0)
            pltpu.sync_copy(ones, dacc.at[idd0], add=True)
            pltpu.sync_copy(dst_h.at[pl.ds(base + (w + 1) * WD, WD)], idd1)
            pltpu.sync_copy(ones, dacc.at[idd1], add=True)
        plsc.subcore_barrier()

        pltpu.sync_copy(dacc.at[pl.ds(r0, RPS)], deg_h.at[cid, pl.ds(r0, RPS)])

    return k(dstp)


def _sc_agg(xchunks, srcp, dstp):
    nchunk = len(xchunks)
    out_t = tuple(jax.ShapeDtypeStruct((NPAD, C), jnp.float32)
                  for _ in range(nchunk))

    @functools.partial(
        pl.kernel, mesh=_MESH, compiler_params=_CP,
        out_type=out_t,
        scratch_types=[
            pltpu.VMEM((KGA, WA), jnp.int32),
            pltpu.VMEM((KGA, WA), jnp.int32),
            pltpu.VMEM((WA, C), jnp.float32),
            pltpu.VMEM((WA, C), jnp.float32),
            pltpu.VMEM((112, C), jnp.float32),
            pltpu.VMEM_SHARED((NPAD, C), jnp.float32),
            pltpu.SemaphoreType.DMA,
            pltpu.SemaphoreType.DMA,
            pltpu.SemaphoreType.DMA,
            pltpu.SemaphoreType.DMA,
        ],
    )
    def k(*refs):
        x_hs = refs[:nchunk]
        src_h, dst_h = refs[nchunk], refs[nchunk + 1]
        out_hs = refs[nchunk + 2:2 * nchunk + 2]
        (sblk, dblk, rows0, rows1, zb, acc,
         gsem0, gsem1, ssem0, ssem1) = refs[2 * nchunk + 2:]
        cid = lax.axis_index("c")
        sid = lax.axis_index("s")
        r0 = sid * RPS

        _fill_zeros_2d(zb, 112, C)

        def windows(x_h):
            @pl.loop(0, GRPA)
            def _(g):
                pltpu.sync_copy(src_h.at[sid, g], sblk)
                pltpu.sync_copy(dst_h.at[sid, g], dblk)

                @pl.loop(0, KGA, step=2)
                def _(j):
                    cg0 = pltpu.async_copy(x_h.at[sblk.at[j]], rows0, gsem0)
                    cg1 = pltpu.async_copy(x_h.at[sblk.at[j + 1]], rows1, gsem1)
                    cg0.wait()
                    cs0 = pltpu.async_copy(rows0, acc.at[dblk.at[j]],
                                           ssem0, add=True)
                    cg1.wait()
                    cs1 = pltpu.async_copy(rows1, acc.at[dblk.at[j + 1]],
                                           ssem1, add=True)
                    cs0.wait()
                    cs1.wait()

        for kc in range(nchunk // 2):
            @pl.loop(0, RPS // 112)
            def _(kk):
                pltpu.sync_copy(zb, acc.at[pl.ds(r0 + kk * 112, 112)])
            plsc.subcore_barrier()

            @pl.when(cid == 0)
            def _():
                windows(x_hs[2 * kc])

            @pl.when(cid == 1)
            def _():
                windows(x_hs[2 * kc + 1])
            plsc.subcore_barrier()

            @pl.when(cid == 0)
            def _():
                pltpu.sync_copy(acc.at[pl.ds(r0, RPS)],
                                out_hs[2 * kc].at[pl.ds(r0, RPS)])

            @pl.when(cid == 1)
            def _():
                pltpu.sync_copy(acc.at[pl.ds(r0, RPS)],
                                out_hs[2 * kc + 1].at[pl.ds(r0, RPS)])

    return k(*xchunks, srcp, dstp)


def _sc_sagg(wrow, srcp, dstp):
    @functools.partial(
        pl.kernel, mesh=_MESH, compiler_params=_CP,
        out_type=jax.ShapeDtypeStruct((NC, NPAD, G), jnp.float32),
        scratch_types=[
            pltpu.VMEM((KGS, WS), jnp.int32),
            pltpu.VMEM((KGS, WS), jnp.int32),
            pltpu.VMEM((WS, G), jnp.float32),
            pltpu.VMEM((WS, G), jnp.float32),
            pltpu.VMEM((112, G), jnp.float32),
            pltpu.VMEM_SHARED((NPAD, G), jnp.float32),
            pltpu.SemaphoreType.DMA,
            pltpu.SemaphoreType.DMA,
            pltpu.SemaphoreType.DMA,
            pltpu.SemaphoreType.DMA,
        ],
    )
    def k(w_h, src_h, dst_h, s_h, sblk, dblk, rows0, rows1,
          zb, acc, gsem0, gsem1, ssem0, ssem1):
        cid = lax.axis_index("c")
        sid = lax.axis_index("s")
        wid = sid * NC + cid
        r0 = sid * RPS

        _fill_zeros_2d(zb, 112, G)

        @pl.loop(0, RPS // 112)
        def _(kk):
            pltpu.sync_copy(zb, acc.at[pl.ds(r0 + kk * 112, 112)])
        plsc.subcore_barrier()

        @pl.loop(0, GRPS)
        def _(g):
            pltpu.sync_copy(src_h.at[wid, g], sblk)
            pltpu.sync_copy(dst_h.at[wid, g], dblk)

            @pl.loop(0, KGS, step=2)
            def _(j):
                cg0 = pltpu.async_copy(w_h.at[dblk.at[j]], rows0, gsem0)
                cg1 = pltpu.async_copy(w_h.at[dblk.at[j + 1]], rows1, gsem1)
                cg0.wait()
                cs0 = pltpu.async_copy(rows0, acc.at[sblk.at[j]],
                                       ssem0, add=True)
                cg1.wait()
                cs1 = pltpu.async_copy(rows1, acc.at[sblk.at[j + 1]],
                                       ssem1, add=True)
                cs0.wait()
                cs1.wait()
        plsc.subcore_barrier()

        pltpu.sync_copy(acc.at[pl.ds(r0, RPS)],
                        s_h.at[cid, pl.ds(r0, RPS)])

    return k(wrow, srcp, dstp)


def _prep_body(deg_r, batch_r, winv_r, wrow_r):
    d = deg_r[0] + deg_r[1]
    winv = 1.0 / jnp.maximum(d, 1.0)
    winv_r[...] = winv
    giota = lax.broadcasted_iota(jnp.int32, (RB, G), 1)
    oh = jnp.where(batch_r[...] == giota, 1.0, 0.0)
    wrow_r[...] = oh * winv


def _tc_prep(deg2, batch_p):
    return pl.pallas_call(
        _prep_body,
        grid=(NBLK,),
        in_specs=[
            pl.BlockSpec((NC, RB, 1), lambda i: (0, i, 0)),
            pl.BlockSpec((RB, 1), lambda i: (i, 0)),
        ],
        out_specs=[
            pl.BlockSpec((RB, 1), lambda i: (i, 0)),
            pl.BlockSpec((RB, G), lambda i: (i, 0)),
        ],
        out_shape=(
            jax.ShapeDtypeStruct((NPAD, 1), jnp.float32),
            jax.ShapeDtypeStruct((NPAD, G), jnp.float32),
        ),
    )(deg2, batch_p)


def _lin1_body(*refs):
    a = refs[:DIN // C]
    x_r, winv_r, wl_r, wr_r, b_r = refs[DIN // C:DIN // C + 5]
    outs = refs[DIN // C + 5:]
    agg = jnp.concatenate([r[...] for r in a], axis=1)
    h = (jnp.dot(agg * winv_r[...], wl_r[...], preferred_element_type=jnp.float32)
         + jnp.dot(x_r[...], wr_r[...], preferred_element_type=jnp.float32)
         + b_r[...])
    h = jnp.maximum(h, 0.0)
    for c in range(DH // C):
        outs[c][...] = h[:, c * C:(c + 1) * C]


def _tc_lin1(aggs, x, winv2d, wlT, wrT, b2d):
    spec_c = pl.BlockSpec((RB, C), lambda i: (i, 0))
    full = lambda s: pl.BlockSpec(s, lambda i: tuple(0 for _ in s))
    return pl.pallas_call(
        _lin1_body,
        grid=(NBLK,),
        in_specs=[spec_c] * (DIN // C) + [
            pl.BlockSpec((RB, DIN), lambda i: (i, 0)),
            pl.BlockSpec((RB, 1), lambda i: (i, 0)),
            full((DIN, DH)), full((DIN, DH)), full((1, DH)),
        ],
        out_specs=[spec_c] * (DH // C),
        out_shape=tuple(jax.ShapeDtypeStruct((NPAD, C), jnp.float32)
                        for _ in range(DH // C)),
    )(*aggs, x, winv2d, wlT, wrT, b2d)


def _lin2_body(*refs):
    nc = DH // C
    aggs, h1s = refs[:nc], refs[nc:2 * nc]
    winv_r, wl_r, wr_r, b_r, out_r = refs[2 * nc:]
    agg = jnp.concatenate([r[...] for r in aggs], axis=1)
    h1 = jnp.concatenate([r[...] for r in h1s], axis=1)
    h = (jnp.dot(agg * winv_r[...], wl_r[...], preferred_element_type=jnp.float32)
         + jnp.dot(h1, wr_r[...], preferred_element_type=jnp.float32)
         + b_r[...])
    out_r[...] = jnp.maximum(h, 0.0)


def _tc_lin2(agg2c, h1c, winv2d, wlT, wrT, b2d):
    spec_c = pl.BlockSpec((RB, C), lambda i: (i, 0))
    full = lambda s: pl.BlockSpec(s, lambda i: tuple(0 for _ in s))
    return pl.pallas_call(
        _lin2_body,
        grid=(NBLK,),
        in_specs=[spec_c] * (2 * (DH // C)) + [
            pl.BlockSpec((RB, 1), lambda i: (i, 0)),
            full((DH, DH)), full((DH, DH)), full((1, DH)),
        ],
        out_specs=pl.BlockSpec((RB, DH), lambda i: (i, 0)),
        out_shape=jax.ShapeDtypeStruct((NPAD, DH), jnp.float32),
    )(*agg2c, *h1c, winv2d, wlT, wrT, b2d)


def _pool_body(h2_r, s_r, batch_r, wl_r, wr_r, b_r, g_r, be_r, out_r,
               ts_ref, tp_ref, cnt_ref):
    i = pl.program_id(0)

    @pl.when(i == 0)
    def _():
        ts_ref[...] = jnp.zeros((G, DH), jnp.float32)
        tp_ref[...] = jnp.zeros((G, DH), jnp.float32)
        cnt_ref[...] = jnp.zeros((1, G), jnp.float32)

    rows = i * RB + lax.broadcasted_iota(jnp.int32, (RB, G), 0)
    valid = rows < N
    s_blk = jnp.where(valid, s_r[0] + s_r[1], 0.0)
    giota = lax.broadcasted_iota(jnp.int32, (RB, G), 1)
    oh = jnp.where((batch_r[...] == giota) & valid, 1.0, 0.0)
    h2 = h2_r[...]
    dn = (((0,), (0,)), ((), ()))
    ts_ref[...] += lax.dot_general(s_blk, h2, dn,
                                   preferred_element_type=jnp.float32)
    tp_ref[...] += lax.dot_general(oh, h2, dn,
                                   preferred_element_type=jnp.float32)
    cnt_ref[...] += jnp.sum(oh, axis=0)[None, :]

    @pl.when(i == NBLK - 1)
    def _():
        cnt = cnt_ref[0, :]
        sums = (jnp.dot(ts_ref[...], wl_r[...],
                        preferred_element_type=jnp.float32)
                + jnp.dot(tp_ref[...], wr_r[...],
                          preferred_element_type=jnp.float32)
                + cnt[:, None] * b_r[...])
        pooled = sums / jnp.maximum(cnt, 1.0)[:, None]
        mu = jnp.mean(pooled, axis=1, keepdims=True)
        var = jnp.mean((pooled - mu) ** 2, axis=1, keepdims=True)
        normed = (pooled - mu) / jnp.sqrt(var + 1e-5)
        out_r[...] = normed * g_r[...] + be_r[...]


def _tc_pool(h2, s2, batch_p, w3lT, w3rT, b3_2d, gamma2d, beta2d):
    full = lambda s: pl.BlockSpec(s, lambda i: tuple(0 for _ in s))
    return pl.pallas_call(
        _pool_body,
        grid=(NBLK,),
        in_specs=[
            pl.BlockSpec((RB, DH), lambda i: (i, 0)),
            pl.BlockSpec((NC, RB, G), lambda i: (0, i, 0)),
            pl.BlockSpec((RB, 1), lambda i: (i, 0)),
            full((DH, DOUT)), full((DH, DOUT)), full((1, DOUT)),
            full((1, DOUT)), full((1, DOUT)),
        ],
        out_specs=full((G, DOUT)),
        out_shape=jax.ShapeDtypeStruct((G, DOUT), jnp.float32),
        scratch_shapes=[
            pltpu.VMEM((G, DH), jnp.float32),
            pltpu.VMEM((G, DH), jnp.float32),
            pltpu.VMEM((1, G), jnp.float32),
        ],
    )(h2, s2, batch_p, w3lT, w3rT, b3_2d, gamma2d, beta2d)


def kernel(x, edge_index, batch, W1l, W1r, b1, W2l, W2r, b2,
           W3l, W3r, b3, gamma, beta):
    src = edge_index[0]
    dst = edge_index[1]
    pad_e = EPAD - E
    srcp = jnp.concatenate([src, jnp.full((pad_e,), N, jnp.int32)])
    dstp = jnp.concatenate([dst, jnp.full((pad_e,), N, jnp.int32)])
    batch_p = jnp.pad(batch, (0, NPAD - N),
                      constant_values=G).reshape(NPAD, 1)
    xp = jnp.pad(x, ((0, NPAD - N), (0, 0)))
    xchunks = [xp[:, c * C:(c + 1) * C] for c in range(DIN // C)]

    srcA = srcp.reshape(NS, GRPA, KGA, WA)
    dstA = dstp.reshape(NS, GRPA, KGA, WA)
    srcS = srcp.reshape(NC * NS, GRPS, KGS, WS)
    dstS = dstp.reshape(NC * NS, GRPS, KGS, WS)

    deg2 = _sc_deg(dstp).reshape(NC, NPAD, 1)
    winv2d, wrow = _tc_prep(deg2, batch_p)

    agg1 = _sc_agg(xchunks, srcA, dstA)
    h1c = _tc_lin1(agg1, xp, winv2d, W1l.T, W1r.T, b1.reshape(1, DH))

    s2 = _sc_sagg(wrow, srcS, dstS)

    agg2 = _sc_agg(list(h1c), srcA, dstA)
    h2 = _tc_lin2(agg2, h1c, winv2d, W2l.T, W2r.T, b2.reshape(1, DH))

    return _tc_pool(h2, s2, batch_p, W3l.T, W3r.T, b3.reshape(1, DOUT),
                    gamma.reshape(1, DOUT), beta.reshape(1, DOUT))
